# baseline (speedup 1.0000x reference)
import jax
import jax.numpy as jnp
from jax.experimental import pallas as pl
from jax.experimental.pallas import tpu as pltpu

_PE_ROWS = 20
_TILE_REPS = 32


def _gather_kernel(pe_ref, idx_ref, o_ref):
    idx = idx_ref[...]
    rows = idx.shape[0]
    iota = jax.lax.broadcasted_iota(jnp.int32, (rows, _PE_ROWS), 1)
    onehot = (idx == iota).astype(jnp.float32)
    o_ref[...] = jnp.dot(onehot, pe_ref[...], preferred_element_type=jnp.float32)


def _add_kernel(x_ref, fpe_ref, o_ref):
    o_ref[...] = x_ref[...] + fpe_ref[...]


def kernel(x, pe, inds):
    batch, seq, d_model = x.shape
    rows_blk = seq * _TILE_REPS
    idx_full = jnp.tile(inds.astype(jnp.int32), _TILE_REPS).reshape(rows_blk, 1)

    fpe_tile = pl.pallas_call(
        _gather_kernel,
        out_shape=jax.ShapeDtypeStruct((rows_blk, d_model), jnp.float32),
    )(pe, idx_full)

    x2 = x.reshape(batch * seq, d_model)
    n_rows = batch * seq
    grid = (n_rows // rows_blk,)
    out2 = pl.pallas_call(
        _add_kernel,
        grid=grid,
        in_specs=[
            pl.BlockSpec((rows_blk, d_model), lambda i: (i, 0)),
            pl.BlockSpec((rows_blk, d_model), lambda i: (0, 0)),
        ],
        out_specs=pl.BlockSpec((rows_blk, d_model), lambda i: (i, 0)),
        out_shape=jax.ShapeDtypeStruct((n_rows, d_model), jnp.float32),
        compiler_params=pltpu.CompilerParams(
            dimension_semantics=("arbitrary",),
        ),
    )(x2, fpe_tile)
    return out2.reshape(batch, seq, d_model)


# speedup vs baseline: 6.2718x; 6.2718x over previous
import jax
import jax.numpy as jnp
from jax.experimental import pallas as pl
from jax.experimental.pallas import tpu as pltpu

_BATCH_BLK = 1024


def _add_kernel(inds_ref, x_ref, pe_ref, o_ref):
    del inds_ref
    o_ref[...] = x_ref[...] + pe_ref[...]


def kernel(x, pe, inds):
    batch, seq, d_model = x.shape
    inds32 = inds.astype(jnp.int32)
    xt = jnp.transpose(x, (1, 0, 2))
    pe3 = pe.reshape(pe.shape[0], 1, d_model)

    grid = (seq, batch // _BATCH_BLK)
    out_t = pl.pallas_call(
        _add_kernel,
        grid_spec=pltpu.PrefetchScalarGridSpec(
            num_scalar_prefetch=1,
            grid=grid,
            in_specs=[
                pl.BlockSpec((1, _BATCH_BLK, d_model), lambda j, i, inds_ref: (j, i, 0)),
                pl.BlockSpec((1, 1, d_model), lambda j, i, inds_ref: (inds_ref[j], 0, 0)),
            ],
            out_specs=pl.BlockSpec((1, _BATCH_BLK, d_model), lambda j, i, inds_ref: (j, i, 0)),
        ),
        out_shape=jax.ShapeDtypeStruct((seq, batch, d_model), jnp.float32),
        compiler_params=pltpu.CompilerParams(
            dimension_semantics=("arbitrary", "arbitrary"),
        ),
    )(inds32, xt, pe3)
    return jnp.transpose(out_t, (1, 0, 2))
